# baseline (speedup 1.0000x reference)
import functools
import jax
import jax.numpy as jnp
from jax import lax
from jax.experimental import pallas as pl
from jax.experimental.pallas import tpu as pltpu
from jax.experimental.pallas import tpu_sc as plsc

NC, NS, L = 2, 16, 16
NW = NC * NS

BS, T, D = 4, 2048, 1024
ROWS = BS * T
ROWS_W = ROWS // NW
CH_R = 16
NSTEP = ROWS_W // CH_R
UNROLL = 8


def _make_sc():
    mesh = plsc.VectorSubcoreMesh(core_axis_name="c", subcore_axis_name="s")

    @functools.partial(
        pl.kernel,
        mesh=mesh,
        out_type=jax.ShapeDtypeStruct((ROWS, D), jnp.float32),
        scratch_types=[
            pltpu.VMEM((2, CH_R, D), jnp.float32),
            pltpu.VMEM((2, CH_R, D), jnp.float32),
            pltpu.VMEM((2, CH_R, D), jnp.float32),
            pltpu.SemaphoreType.DMA((2,)),
            pltpu.SemaphoreType.DMA((2,)),
            pltpu.SemaphoreType.DMA((2,)),
        ],
    )
    def sc_add(x_hbm, w_hbm, o_hbm, xb, wb, ob, sx, sw, so):
        wid = lax.axis_index("s") * NC + lax.axis_index("c")
        xrow = wid * ROWS_W
        wrow = lax.rem(wid, T // ROWS_W) * ROWS_W

        def start_in(slot, i):
            pltpu.make_async_copy(
                x_hbm.at[pl.ds(xrow + i * CH_R, CH_R)], xb.at[slot], sx.at[slot]
            ).start()
            pltpu.make_async_copy(
                w_hbm.at[pl.ds(wrow + i * CH_R, CH_R)], wb.at[slot], sw.at[slot]
            ).start()

        def wait_in(slot):
            pltpu.make_async_copy(
                x_hbm.at[pl.ds(xrow, CH_R)], xb.at[slot], sx.at[slot]
            ).wait()
            pltpu.make_async_copy(
                w_hbm.at[pl.ds(wrow, CH_R)], wb.at[slot], sw.at[slot]
            ).wait()

        def start_out(slot, i):
            pltpu.make_async_copy(
                ob.at[slot], o_hbm.at[pl.ds(xrow + i * CH_R, CH_R)], so.at[slot]
            ).start()

        def wait_out(slot):
            pltpu.make_async_copy(
                ob.at[slot], o_hbm.at[pl.ds(xrow, CH_R)], so.at[slot]
            ).wait()

        start_in(0, 0)
        start_in(1, 1)

        def pair(ip, carry):
            for b in range(2):
                i = ip * 2 + b
                wait_in(b)

                @pl.when(i >= 2)
                def _():
                    wait_out(b)

                def vbody(k, c):
                    for u in range(UNROLL):
                        sl = pl.ds((k * UNROLL + u) * L, L)
                        for r in range(CH_R):
                            ob[b, r, sl] = xb[b, r, sl] + wb[b, r, sl]
                    return c

                lax.fori_loop(0, D // (UNROLL * L), vbody, 0)
                start_out(b, i)

                @pl.when(i + 2 < NSTEP)
                def _():
                    start_in(b, i + 2)
            return carry

        lax.fori_loop(0, NSTEP // 2, pair, 0)
        wait_out(0)
        wait_out(1)

    return sc_add


_sc_add = _make_sc()


def kernel(inputs, embed_weight):
    bs, t, d = inputs.shape
    out = _sc_add(inputs.reshape(bs * t, d), embed_weight)
    return out.reshape(bs, t, d)


# speedup vs baseline: 1.1953x; 1.1953x over previous
import functools
import jax
import jax.numpy as jnp
from jax import lax
from jax.experimental import pallas as pl
from jax.experimental.pallas import tpu as pltpu
from jax.experimental.pallas import tpu_sc as plsc

NC, NS, L = 2, 16, 16
NW = NC * NS

BS, T, D = 4, 2048, 1024
SC_BATCHES = 1
TC_BATCHES = BS - SC_BATCHES
SC_ROW0 = TC_BATCHES * T
ROWS_W = SC_BATCHES * T // NW
CH_R = 16
NSTEP = ROWS_W // CH_R
UNROLL = 8


def _make_sc():
    mesh = plsc.VectorSubcoreMesh(core_axis_name="c", subcore_axis_name="s")

    @functools.partial(
        pl.kernel,
        mesh=mesh,
        out_type=jax.ShapeDtypeStruct((SC_BATCHES * T, D), jnp.float32),
        scratch_types=[
            pltpu.VMEM((2, CH_R, D), jnp.float32),
            pltpu.VMEM((2, CH_R, D), jnp.float32),
            pltpu.VMEM((2, CH_R, D), jnp.float32),
            pltpu.SemaphoreType.DMA((2,)),
            pltpu.SemaphoreType.DMA((2,)),
            pltpu.SemaphoreType.DMA((2,)),
        ],
    )
    def sc_add(x_hbm, w_hbm, o_hbm, xb, wb, ob, sx, sw, so):
        wid = lax.axis_index("s") * NC + lax.axis_index("c")
        orow = wid * ROWS_W
        xrow = SC_ROW0 + orow
        wrow = orow

        def start_in(slot, i):
            pltpu.make_async_copy(
                x_hbm.at[pl.ds(xrow + i * CH_R, CH_R)], xb.at[slot], sx.at[slot]
            ).start()
            pltpu.make_async_copy(
                w_hbm.at[pl.ds(wrow + i * CH_R, CH_R)], wb.at[slot], sw.at[slot]
            ).start()

        def wait_in(slot):
            pltpu.make_async_copy(
                x_hbm.at[pl.ds(xrow, CH_R)], xb.at[slot], sx.at[slot]
            ).wait()
            pltpu.make_async_copy(
                w_hbm.at[pl.ds(wrow, CH_R)], wb.at[slot], sw.at[slot]
            ).wait()

        def start_out(slot, i):
            pltpu.make_async_copy(
                ob.at[slot], o_hbm.at[pl.ds(orow + i * CH_R, CH_R)], so.at[slot]
            ).start()

        def wait_out(slot):
            pltpu.make_async_copy(
                ob.at[slot], o_hbm.at[pl.ds(orow, CH_R)], so.at[slot]
            ).wait()

        start_in(0, 0)
        start_in(1, 1)

        def pair(ip, carry):
            for b in range(2):
                i = ip * 2 + b
                wait_in(b)

                @pl.when(i >= 2)
                def _():
                    wait_out(b)

                def vbody(k, c):
                    for u in range(UNROLL):
                        sl = pl.ds((k * UNROLL + u) * L, L)
                        for r in range(CH_R):
                            ob[b, r, sl] = xb[b, r, sl] + wb[b, r, sl]
                    return c

                lax.fori_loop(0, D // (UNROLL * L), vbody, 0)
                start_out(b, i)

                @pl.when(i + 2 < NSTEP)
                def _():
                    start_in(b, i + 2)
            return carry

        lax.fori_loop(0, NSTEP // 2, pair, 0)
        wait_out(0)
        wait_out(1)

    return sc_add


_sc_add = _make_sc()


def _tc_add_kernel(x_ref, w_ref, o_ref):
    o_ref[...] = x_ref[...] + w_ref[...]


def _tc_add(x2, embed_weight):
    return pl.pallas_call(
        _tc_add_kernel,
        grid=(1, TC_BATCHES),
        in_specs=[
            pl.BlockSpec((T, D), lambda t, b: (b, 0)),
            pl.BlockSpec((T, D), lambda t, b: (0, 0)),
        ],
        out_specs=pl.BlockSpec((T, D), lambda t, b: (b, 0)),
        out_shape=jax.ShapeDtypeStruct((TC_BATCHES * T, D), x2.dtype),
    )(x2, embed_weight)


def kernel(inputs, embed_weight):
    bs, t, d = inputs.shape
    x2 = inputs.reshape(bs * t, d)
    out_sc = _sc_add(x2, embed_weight)
    out_tc = _tc_add(x2, embed_weight)
    return jnp.concatenate([out_tc, out_sc], axis=0).reshape(bs, t, d)


# speedup vs baseline: 3.4836x; 2.9145x over previous
import jax
import jax.numpy as jnp
from jax.experimental import pallas as pl


def _add_kernel(x_ref, w_ref, o_ref):
    i = pl.program_id(0)
    off = (i % 2) * 1024
    o_ref[...] = x_ref[...] + w_ref[pl.ds(off, 1024), :]


def kernel(inputs, embed_weight):
    bs, T, D = inputs.shape
    blk = 1024
    n = bs * T // blk
    x2 = inputs.reshape(bs * T, D)
    out = pl.pallas_call(
        _add_kernel,
        grid=(n,),
        in_specs=[
            pl.BlockSpec((blk, D), lambda i: (i, 0)),
            pl.BlockSpec((T, D), lambda i: (0, 0)),
        ],
        out_specs=pl.BlockSpec((blk, D), lambda i: (i, 0)),
        out_shape=jax.ShapeDtypeStruct((bs * T, D), inputs.dtype),
    )(x2, embed_weight)
    return out.reshape(bs, T, D)


# speedup vs baseline: 3.7047x; 1.0635x over previous
import jax
import jax.numpy as jnp
from jax.experimental import pallas as pl


def _add_kernel(x_ref, w_ref, o_ref):
    o_ref[...] = x_ref[...] + w_ref[...]


def kernel(inputs, embed_weight):
    bs, T, D = inputs.shape
    blk = 2048
    nt = T // blk
    x2 = inputs.reshape(bs * T, D)
    out = pl.pallas_call(
        _add_kernel,
        grid=(nt, bs),
        in_specs=[
            pl.BlockSpec((blk, D), lambda t, b: (b * nt + t, 0)),
            pl.BlockSpec((blk, D), lambda t, b: (t, 0)),
        ],
        out_specs=pl.BlockSpec((blk, D), lambda t, b: (b * nt + t, 0)),
        out_shape=jax.ShapeDtypeStruct((bs * T, D), inputs.dtype),
    )(x2, embed_weight)
    return out.reshape(bs, T, D)
